# baseline (speedup 1.0000x reference)
import functools

import jax
import jax.numpy as jnp
from jax import lax
from jax.experimental import pallas as pl
from jax.experimental.pallas import tpu as pltpu
from jax.experimental.pallas import tpu_sc as plsc

_F32 = jnp.float32


def _dot(a, b):
    return jnp.dot(a, b, preferred_element_type=_F32)


def _proj_body(xu_ref, xi_ref, WpuT_ref, bpu_ref, WpiT_ref, bpi_ref,
               WrrT_ref, WrbT_ref, WrsT_ref,
               xu_lo_ref, xu_hi_ref, xi_lo_ref, xi_hi_ref,
               rr_ref, rb_ref, rs_ref):
    xu = jnp.maximum(_dot(xu_ref[...], WpuT_ref[...]) + bpu_ref[...], 0.0)
    xi = jnp.maximum(_dot(xi_ref[...], WpiT_ref[...]) + bpi_ref[...], 0.0)
    Hh = xu.shape[1] // 2
    xu_lo_ref[...] = xu[:, :Hh]
    xu_hi_ref[...] = xu[:, Hh:]
    xi_lo_ref[...] = xi[:, :Hh]
    xi_hi_ref[...] = xi[:, Hh:]
    rr_ref[...] = _dot(xi, WrrT_ref[...])
    rb_ref[...] = _dot(xu, WrbT_ref[...])
    rs_ref[...] = _dot(xi, WrsT_ref[...])


def _make_segsum(NA, SL, E_pw, CH, K, Hh):
    mesh = plsc.VectorSubcoreMesh(core_axis_name="c", subcore_axis_name="s")

    @functools.partial(
        pl.kernel,
        out_type=(
            jax.ShapeDtypeStruct((NA, Hh), _F32),
            jax.ShapeDtypeStruct((NA, Hh), _F32),
            jax.ShapeDtypeStruct((NA,), _F32),
            jax.ShapeDtypeStruct((NA,), _F32),
        ),
        mesh=mesh,
        scratch_types=[
            pltpu.VMEM((K,), jnp.int32),
            pltpu.VMEM((K,), jnp.int32),
            pltpu.VMEM((K,), jnp.int32),
            pltpu.VMEM((K,), jnp.int32),
            pltpu.VMEM((K,), jnp.int32),
            pltpu.VMEM((K,), jnp.int32),
            pltpu.VMEM((K, Hh), _F32),
            pltpu.VMEM((K, Hh), _F32),
            pltpu.VMEM((K, Hh), _F32),
            pltpu.VMEM((128,), _F32),
            pltpu.VMEM_SHARED((NA, Hh), _F32),
            pltpu.VMEM_SHARED((NA,), _F32),
            pltpu.SemaphoreType.DMA,
            pltpu.SemaphoreType.DMA,
            pltpu.SemaphoreType.DMA,
            pltpu.SemaphoreType.DMA,
            pltpu.SemaphoreType.DMA,
            pltpu.SemaphoreType.DMA,
        ],
    )
    def k(lo_hbm, hi_hbm, ei_hbm, z2_hbm, z1_hbm,
          sum_lo_hbm, sum_hi_hbm, cnt0_hbm, cnt1_hbm,
          sidx0, sidx1, sidx2, didx0, didx1, didx2,
          rows0, rows1, rows2, ones_pad,
          accum, cacc, g0, g1, g2, s0, s1, s2):
        c = lax.axis_index("c")
        s = lax.axis_index("s")
        sidx = (sidx0, sidx1, sidx2)
        didx = (didx0, didx1, didx2)
        rows = (rows0, rows1, rows2)
        gsem = (g0, g1, g2)
        ssem = (s0, s1, s2)
        ones = ones_pad.at[pl.ds(0, K)]
        rbase = s * SL
        pltpu.sync_copy(z2_hbm.at[pl.ds(rbase, SL)], accum.at[pl.ds(rbase, SL)])
        pltpu.sync_copy(z1_hbm.at[pl.ds(rbase, SL)], cacc.at[pl.ds(rbase, SL)])
        for j in range(128 // 16):
            ones_pad[pl.ds(j * 16, 16)] = jnp.full((16,), 1.0, _F32)
        plsc.subcore_barrier()

        ebase = s * E_pw

        Epad = E_pw * 16

        def idx_load(i, b):
            off = ebase + i * K
            pltpu.sync_copy(ei_hbm.at[pl.ds(off, K)], sidx[b])
            pltpu.sync_copy(ei_hbm.at[pl.ds(Epad + off, K)], didx[b])

        def gstart(b):
            @pl.when(c == 0)
            def _():
                pltpu.async_copy(lo_hbm.at[sidx[b]], rows[b], gsem[b])

            @pl.when(c == 1)
            def _():
                pltpu.async_copy(hi_hbm.at[sidx[b]], rows[b], gsem[b])

        def gwait(b):
            @pl.when(c == 0)
            def _():
                pltpu.make_async_copy(lo_hbm.at[sidx[b]], rows[b],
                                      gsem[b]).wait()

            @pl.when(c == 1)
            def _():
                pltpu.make_async_copy(hi_hbm.at[sidx[b]], rows[b],
                                      gsem[b]).wait()

        def sstart(i, b):
            pltpu.async_copy(rows[b], accum.at[didx[b]], ssem[b], add=True)

            @pl.when(c == i % 2)
            def _():
                pltpu.async_copy(ones, cacc.at[didx[b]], ssem[b], add=True)

        def swait(i, b):
            pltpu.make_async_copy(rows[b], accum.at[didx[b]],
                                  ssem[b]).wait()

            @pl.when(c == i % 2)
            def _():
                pltpu.make_async_copy(ones, cacc.at[didx[b]],
                                      ssem[b]).wait()

        for t in range(min(2, CH)):
            idx_load(t, t)
            gstart(t)

        def consume(i, b):
            gwait(b)
            sstart(i, b)
            b2 = (b + 2) % 3

            @pl.when(i + 2 < CH)
            def _():
                @pl.when(i - 1 >= 0)
                def _():
                    swait(i - 1, b2)

                idx_load(i + 2, b2)
                gstart(b2)

        def triple(j, carry):
            i0 = 3 * j
            for t in range(3):
                @pl.when(i0 + t < CH)
                def _(t=t):
                    consume(i0 + t, t)

            return carry

        lax.fori_loop(0, (CH + 2) // 3, triple, 0)

        for t in range(3):
            if CH > t:
                i_last = CH - 1 - ((CH - 1 - t) % 3)
                swait(i_last, t)

        plsc.subcore_barrier()

        @pl.when(c == 0)
        def _():
            pltpu.sync_copy(accum.at[pl.ds(rbase, SL)],
                            sum_lo_hbm.at[pl.ds(rbase, SL)])
            pltpu.sync_copy(cacc.at[pl.ds(rbase, SL)],
                            cnt0_hbm.at[pl.ds(rbase, SL)])

        @pl.when(c == 1)
        def _():
            pltpu.sync_copy(accum.at[pl.ds(rbase, SL)],
                            sum_hi_hbm.at[pl.ds(rbase, SL)])
            pltpu.sync_copy(cacc.at[pl.ds(rbase, SL)],
                            cnt1_hbm.at[pl.ds(rbase, SL)])

    return k


def _sage_body(NREAL, with_attn,
               slo_ref, shi_ref, cnt0_ref, cnt1_ref, r_ref,
               WlT_lo_ref, WlT_hi_ref, bl_ref,
               Wa1T_ref, ba1_ref, Wa2T_ref,
               m_ref, acc_ref):
    i = pl.program_id(0)
    RB = m_ref.shape[0]
    t = _dot(slo_ref[...], WlT_lo_ref[...]) + _dot(shi_ref[...], WlT_hi_ref[...])
    inv = 1.0 / jnp.maximum(cnt0_ref[...] + cnt1_ref[...], 1.0)
    m = jnp.maximum(t * inv + bl_ref[...] + r_ref[...], 0.0)
    m_ref[...] = m

    rows = lax.broadcasted_iota(jnp.int32, (RB, 1), 0) + i * RB
    maskf = jnp.where(rows < NREAL, 1.0, 0.0).astype(_F32)
    H = m.shape[1]
    cs = jnp.sum(m * maskf, axis=0, keepdims=True)
    if with_attn:
        a = _dot(jnp.tanh(_dot(m, Wa1T_ref[...]) + ba1_ref[...]),
                 Wa2T_ref[...])
        w = jnp.sum(a * maskf)
        col = lax.broadcasted_iota(jnp.int32, (1, H), 1)
        wrow = jnp.where(col == 0, w, 0.0)
    else:
        wrow = jnp.zeros((1, H), _F32)
    upd = jnp.concatenate([cs, wrow, jnp.zeros((6, H), _F32)], axis=0)

    @pl.when(i == 0)
    def _():
        acc_ref[...] = jnp.zeros_like(acc_ref)

    acc_ref[...] += upd


def _combine_body(NREAL, m_rates_ref, m_sim_ref,
                  acc_r_ref, acc_b_ref, acc_s_ref,
                  out_item_ref, pooled_ref):
    i = pl.program_id(0)
    H = acc_r_ref.shape[1]
    col = lax.broadcasted_iota(jnp.int32, (1, H), 1)
    wa = jnp.sum(jnp.where(col == 0, acc_r_ref[1:2, :], 0.0)) / NREAL
    ws = jnp.sum(jnp.where(col == 0, acc_s_ref[1:2, :], 0.0)) / NREAL
    m = jnp.maximum(wa, ws)
    ea = jnp.exp(wa - m)
    es = jnp.exp(ws - m)
    tot = ea + es
    b0 = ea / tot
    b1 = es / tot
    out_item_ref[...] = b0 * m_rates_ref[...] + b1 * m_sim_ref[...]

    @pl.when(i == 0)
    def _():
        pooled_ref[...] = jnp.concatenate(
            [acc_b_ref[0:1, :], b0 * acc_r_ref[0:1, :] + b1 * acc_s_ref[0:1, :]],
            axis=0) / NREAL


def kernel(x_user, x_item, ei_rates, ei_rated_by, ei_similar,
           Wp_user, bp_user, Wp_item, bp_item,
           Wl_rates, bl_rates, Wr_rates,
           Wl_rb, bl_rb, Wr_rb,
           Wl_sim, bl_sim, Wr_sim,
           Wa1, ba1, Wa2):
    NREAL, D = x_user.shape
    H = Wp_user.shape[0]
    Hh = H // 2
    RB = 512
    nblk = -(-NREAL // RB)
    NA = nblk * RB
    NS = 16
    K = 120
    E = ei_rates.shape[1]
    CH = -(-E // (NS * K))
    Epad = CH * NS * K
    E_pw = Epad // NS
    SL = NA // NS

    f32 = _F32
    pad_n = NA - NREAL

    xu_p = jnp.concatenate([x_user.astype(f32), jnp.zeros((pad_n, D), f32)], axis=0)
    xi_p = jnp.concatenate([x_item.astype(f32), jnp.zeros((pad_n, D), f32)], axis=0)

    pe = Epad - E
    epad = jnp.concatenate(
        [jnp.zeros((1, pe), jnp.int32), jnp.full((1, pe), NREAL, jnp.int32)],
        axis=0)

    def edges(ei):
        return jnp.concatenate([ei.astype(jnp.int32), epad],
                               axis=1).reshape(2 * Epad)

    ei_r = edges(ei_rates)
    ei_b = edges(ei_rated_by)
    ei_s = edges(ei_similar)

    WpuT = Wp_user.T.astype(f32)
    WpiT = Wp_item.T.astype(f32)
    bpu = bp_user.reshape(1, H).astype(f32)
    bpi = bp_item.reshape(1, H).astype(f32)
    WrrT = Wr_rates.T.astype(f32)
    WrbT = Wr_rb.T.astype(f32)
    WrsT = Wr_sim.T.astype(f32)

    full = lambda shape: pl.BlockSpec(shape, lambda i: tuple(0 for _ in shape))
    rowblk = lambda w: pl.BlockSpec((RB, w), lambda i: (i, 0))

    xu_lo, xu_hi, xi_lo, xi_hi, rr, rb, rs = pl.pallas_call(
        _proj_body,
        grid=(nblk,),
        in_specs=[rowblk(D), rowblk(D),
                  full((D, H)), full((1, H)), full((D, H)), full((1, H)),
                  full((H, H)), full((H, H)), full((H, H))],
        out_specs=[rowblk(Hh), rowblk(Hh), rowblk(Hh), rowblk(Hh),
                   rowblk(H), rowblk(H), rowblk(H)],
        out_shape=[jax.ShapeDtypeStruct((NA, Hh), f32),
                   jax.ShapeDtypeStruct((NA, Hh), f32),
                   jax.ShapeDtypeStruct((NA, Hh), f32),
                   jax.ShapeDtypeStruct((NA, Hh), f32),
                   jax.ShapeDtypeStruct((NA, H), f32),
                   jax.ShapeDtypeStruct((NA, H), f32),
                   jax.ShapeDtypeStruct((NA, H), f32)],
    )(xu_p, xi_p, WpuT, bpu, WpiT, bpi, WrrT, WrbT, WrsT)

    z2 = jnp.zeros((NA, Hh), f32)
    z1 = jnp.zeros((NA,), f32)
    segsum = _make_segsum(NA, SL, E_pw, CH, K, Hh)

    slo_r, shi_r, cnt_r0, cnt_r1 = segsum(xu_lo, xu_hi, ei_r, z2, z1)
    slo_b, shi_b, cnt_b0, cnt_b1 = segsum(xi_lo, xi_hi, ei_b, z2, z1)
    slo_s, shi_s, cnt_s0, cnt_s1 = segsum(xi_lo, xi_hi, ei_s, z2, z1)

    WlrT = Wl_rates.T.astype(f32)
    WlbT = Wl_rb.T.astype(f32)
    WlsT = Wl_sim.T.astype(f32)
    blr = bl_rates.reshape(1, H).astype(f32)
    blb = bl_rb.reshape(1, H).astype(f32)
    bls = bl_sim.reshape(1, H).astype(f32)
    Wa1T = Wa1.T.astype(f32)
    ba1r = ba1.reshape(1, H).astype(f32)
    Wa2T = Wa2.T.astype(f32)

    def sage_tc(with_attn, slo, shi, c0, c1, r, WlT, bl):
        return pl.pallas_call(
            functools.partial(_sage_body, NREAL, with_attn),
            grid=(nblk,),
            in_specs=[rowblk(Hh), rowblk(Hh), rowblk(1), rowblk(1),
                      rowblk(H),
                      full((Hh, H)), full((Hh, H)), full((1, H)),
                      full((H, H)), full((1, H)), full((H, 1))],
            out_specs=[rowblk(H), pl.BlockSpec((8, H), lambda i: (0, 0))],
            out_shape=[jax.ShapeDtypeStruct((NA, H), f32),
                       jax.ShapeDtypeStruct((8, H), f32)],
        )(slo, shi, c0.reshape(NA, 1), c1.reshape(NA, 1), r,
          WlT[:Hh], WlT[Hh:], bl, Wa1T, ba1r, Wa2T)

    m_rates, acc_r = sage_tc(True, slo_r, shi_r, cnt_r0, cnt_r1, rr, WlrT, blr)
    out_user_f, acc_b = sage_tc(False, slo_b, shi_b, cnt_b0, cnt_b1, rb,
                                WlbT, blb)
    m_sim, acc_s = sage_tc(True, slo_s, shi_s, cnt_s0, cnt_s1, rs, WlsT, bls)

    out_item_f, pooled2 = pl.pallas_call(
        functools.partial(_combine_body, float(NREAL)),
        grid=(nblk,),
        in_specs=[rowblk(H), rowblk(H),
                  pl.BlockSpec((8, H), lambda i: (0, 0)),
                  pl.BlockSpec((8, H), lambda i: (0, 0)),
                  pl.BlockSpec((8, H), lambda i: (0, 0))],
        out_specs=[rowblk(H), pl.BlockSpec((2, H), lambda i: (0, 0))],
        out_shape=[jax.ShapeDtypeStruct((NA, H), f32),
                   jax.ShapeDtypeStruct((2, H), f32)],
    )(m_rates, m_sim, acc_r, acc_b, acc_s)

    pooled = jnp.concatenate([pooled2[0], pooled2[1]], axis=0)
    return (pooled, out_user_f[:NREAL], out_item_f[:NREAL])


# speedup vs baseline: 1.0591x; 1.0591x over previous
import functools

import jax
import jax.numpy as jnp
from jax import lax
from jax.experimental import pallas as pl
from jax.experimental.pallas import tpu as pltpu
from jax.experimental.pallas import tpu_sc as plsc

_F32 = jnp.float32


def _dot(a, b):
    return jnp.dot(a, b, preferred_element_type=_F32)


def _proj_body(xu_ref, xi_ref, WpuT_ref, bpu_ref, WpiT_ref, bpi_ref,
               WrrT_ref, WrbT_ref, WrsT_ref,
               xu_lo_ref, xu_hi_ref, xi_lo_ref, xi_hi_ref,
               rr_ref, rb_ref, rs_ref):
    xu = jnp.maximum(_dot(xu_ref[...], WpuT_ref[...]) + bpu_ref[...], 0.0)
    xi = jnp.maximum(_dot(xi_ref[...], WpiT_ref[...]) + bpi_ref[...], 0.0)
    Hh = xu.shape[1] // 2
    xu_lo_ref[...] = xu[:, :Hh]
    xu_hi_ref[...] = xu[:, Hh:]
    xi_lo_ref[...] = xi[:, :Hh]
    xi_hi_ref[...] = xi[:, Hh:]
    rr_ref[...] = _dot(xi, WrrT_ref[...])
    rb_ref[...] = _dot(xu, WrbT_ref[...])
    rs_ref[...] = _dot(xi, WrsT_ref[...])


def _make_segsum(NA, SL, E_pw, CH, K, Hh):
    mesh = plsc.VectorSubcoreMesh(core_axis_name="c", subcore_axis_name="s")

    @functools.partial(
        pl.kernel,
        out_type=(
            jax.ShapeDtypeStruct((NA, Hh), _F32),
            jax.ShapeDtypeStruct((NA, Hh), _F32),
            jax.ShapeDtypeStruct((NA,), _F32),
        ),
        mesh=mesh,
        scratch_types=[
            pltpu.VMEM((K,), jnp.int32),
            pltpu.VMEM((K,), jnp.int32),
            pltpu.VMEM((K,), jnp.int32),
            pltpu.VMEM((K,), jnp.int32),
            pltpu.VMEM((K,), jnp.int32),
            pltpu.VMEM((K,), jnp.int32),
            pltpu.VMEM((K, Hh), _F32),
            pltpu.VMEM((K, Hh), _F32),
            pltpu.VMEM((K, Hh), _F32),
            pltpu.VMEM((128,), _F32),
            pltpu.VMEM_SHARED((NA, Hh), _F32),
            pltpu.VMEM_SHARED((NA,), _F32),
            pltpu.SemaphoreType.DMA,
            pltpu.SemaphoreType.DMA,
            pltpu.SemaphoreType.DMA,
            pltpu.SemaphoreType.DMA,
            pltpu.SemaphoreType.DMA,
            pltpu.SemaphoreType.DMA,
        ],
    )
    def k(lo_hbm, hi_hbm, ei_hbm, z2_hbm, z1_hbm,
          sum_lo_hbm, sum_hi_hbm, cnt_hbm,
          sidx0, sidx1, sidx2, didx0, didx1, didx2,
          rows0, rows1, rows2, ones_pad,
          accum, cacc, g0, g1, g2, s0, s1, s2):
        c = lax.axis_index("c")
        s = lax.axis_index("s")
        sidx = (sidx0, sidx1, sidx2)
        didx = (didx0, didx1, didx2)
        rows = (rows0, rows1, rows2)
        gsem = (g0, g1, g2)
        ssem = (s0, s1, s2)
        ones = ones_pad.at[pl.ds(0, K)]
        rbase = s * SL
        pltpu.sync_copy(z2_hbm.at[pl.ds(rbase, SL)], accum.at[pl.ds(rbase, SL)])

        @pl.when(c == 0)
        def _():
            pltpu.sync_copy(z1_hbm.at[pl.ds(rbase, SL)],
                            cacc.at[pl.ds(rbase, SL)])

        for j in range(128 // 16):
            ones_pad[pl.ds(j * 16, 16)] = jnp.full((16,), 1.0, _F32)
        plsc.subcore_barrier()

        ebase = s * E_pw
        Epad = E_pw * 16

        def gstart(b):
            @pl.when(c == 0)
            def _():
                pltpu.async_copy(lo_hbm.at[sidx[b]], rows[b], gsem[b])

            @pl.when(c == 1)
            def _():
                pltpu.async_copy(hi_hbm.at[sidx[b]], rows[b], gsem[b])

        def gwait(b):
            @pl.when(c == 0)
            def _():
                pltpu.make_async_copy(lo_hbm.at[sidx[b]], rows[b],
                                      gsem[b]).wait()

            @pl.when(c == 1)
            def _():
                pltpu.make_async_copy(hi_hbm.at[sidx[b]], rows[b],
                                      gsem[b]).wait()

        def sstart(b):
            pltpu.async_copy(rows[b], accum.at[didx[b]], ssem[b], add=True)

            @pl.when(c == 0)
            def _():
                pltpu.async_copy(ones, cacc.at[didx[b]], ssem[b], add=True)

        def swait(b):
            pltpu.make_async_copy(rows[b], accum.at[didx[b]],
                                  ssem[b]).wait()

            @pl.when(c == 0)
            def _():
                pltpu.make_async_copy(ones, cacc.at[didx[b]],
                                      ssem[b]).wait()

        for t in range(min(2, CH)):
            off = ebase + t * K
            pltpu.sync_copy(ei_hbm.at[pl.ds(off, K)], sidx[t])
            pltpu.sync_copy(ei_hbm.at[pl.ds(Epad + off, K)], didx[t])
            gstart(t)

        def consume(i, b):
            gwait(b)
            sstart(b)
            b2 = (b + 2) % 3

            @pl.when(i + 2 < CH)
            def _():
                off = ebase + (i + 2) * K
                pltpu.sync_copy(ei_hbm.at[pl.ds(off, K)], sidx[b2])

                @pl.when(i - 1 >= 0)
                def _():
                    swait(b2)

                pltpu.sync_copy(ei_hbm.at[pl.ds(Epad + off, K)], didx[b2])
                gstart(b2)

        def triple(j, carry):
            i0 = 3 * j
            for t in range(3):
                @pl.when(i0 + t < CH)
                def _(t=t):
                    consume(i0 + t, t)

            return carry

        lax.fori_loop(0, (CH + 2) // 3, triple, 0)

        for t in range(3):
            if CH > t:
                swait(t)

        plsc.subcore_barrier()

        @pl.when(c == 0)
        def _():
            pltpu.sync_copy(accum.at[pl.ds(rbase, SL)],
                            sum_lo_hbm.at[pl.ds(rbase, SL)])
            pltpu.sync_copy(cacc.at[pl.ds(rbase, SL)],
                            cnt_hbm.at[pl.ds(rbase, SL)])

        @pl.when(c == 1)
        def _():
            pltpu.sync_copy(accum.at[pl.ds(rbase, SL)],
                            sum_hi_hbm.at[pl.ds(rbase, SL)])

    return k


def _post_body(NREAL,
               slo_r_ref, shi_r_ref, cnt_r_ref,
               slo_b_ref, shi_b_ref, cnt_b_ref,
               slo_s_ref, shi_s_ref, cnt_s_ref,
               rr_ref, rb_ref, rs_ref,
               WlrT_lo_ref, WlrT_hi_ref, blr_ref,
               WlbT_lo_ref, WlbT_hi_ref, blb_ref,
               WlsT_lo_ref, WlsT_hi_ref, bls_ref,
               Wa1T_ref, ba1_ref, Wa2T_ref,
               out_user_ref, m_rates_ref, m_sim_ref, acc_ref):
    i = pl.program_id(0)
    RB = m_rates_ref.shape[0]

    def sage(slo_ref, shi_ref, cnt_ref, wt_lo_ref, wt_hi_ref, bl_ref, r_ref):
        t = _dot(slo_ref[...], wt_lo_ref[...]) + _dot(shi_ref[...], wt_hi_ref[...])
        inv = 1.0 / jnp.maximum(cnt_ref[...], 1.0)
        return jnp.maximum(t * inv + bl_ref[...] + r_ref[...], 0.0)

    m_rates = sage(slo_r_ref, shi_r_ref, cnt_r_ref, WlrT_lo_ref, WlrT_hi_ref,
                   blr_ref, rr_ref)
    m_rb = sage(slo_b_ref, shi_b_ref, cnt_b_ref, WlbT_lo_ref, WlbT_hi_ref,
                blb_ref, rb_ref)
    m_sim = sage(slo_s_ref, shi_s_ref, cnt_s_ref, WlsT_lo_ref, WlsT_hi_ref,
                 bls_ref, rs_ref)
    out_user_ref[...] = m_rb
    m_rates_ref[...] = m_rates
    m_sim_ref[...] = m_sim

    rows = lax.broadcasted_iota(jnp.int32, (RB, 1), 0) + i * RB
    valid = rows < NREAL

    a_rates = _dot(jnp.tanh(_dot(m_rates, Wa1T_ref[...]) + ba1_ref[...]),
                   Wa2T_ref[...])
    a_sim = _dot(jnp.tanh(_dot(m_sim, Wa1T_ref[...]) + ba1_ref[...]),
                 Wa2T_ref[...])

    cs_rb = jnp.sum(jnp.where(valid, m_rb, 0.0), axis=0, keepdims=True)
    cs_rates = jnp.sum(jnp.where(valid, m_rates, 0.0), axis=0, keepdims=True)
    cs_sim = jnp.sum(jnp.where(valid, m_sim, 0.0), axis=0, keepdims=True)
    wa = jnp.sum(jnp.where(valid, a_rates, 0.0))
    ws = jnp.sum(jnp.where(valid, a_sim, 0.0))
    col = lax.broadcasted_iota(jnp.int32, (1, cs_rb.shape[1]), 1)
    wrow = jnp.where(col == 0, wa, 0.0) + jnp.where(col == 1, ws, 0.0)
    upd = jnp.concatenate(
        [cs_rb, cs_rates, cs_sim, wrow,
         jnp.zeros((4, cs_rb.shape[1]), _F32)], axis=0)

    @pl.when(i == 0)
    def _():
        acc_ref[...] = jnp.zeros_like(acc_ref)

    acc_ref[...] += upd


def _combine_body(NREAL, m_rates_ref, m_sim_ref, acc_ref,
                  out_item_ref, pooled_ref):
    i = pl.program_id(0)
    acc = acc_ref[...]
    H = acc.shape[1]
    col = lax.broadcasted_iota(jnp.int32, (1, H), 1)
    wrow = acc[3:4, :]
    wa = jnp.sum(jnp.where(col == 0, wrow, 0.0)) / NREAL
    ws = jnp.sum(jnp.where(col == 1, wrow, 0.0)) / NREAL
    m = jnp.maximum(wa, ws)
    ea = jnp.exp(wa - m)
    es = jnp.exp(ws - m)
    tot = ea + es
    b0 = ea / tot
    b1 = es / tot
    out_item_ref[...] = b0 * m_rates_ref[...] + b1 * m_sim_ref[...]

    @pl.when(i == 0)
    def _():
        pooled_ref[...] = jnp.concatenate(
            [acc[0:1, :], b0 * acc[1:2, :] + b1 * acc[2:3, :]],
            axis=0) / NREAL


def kernel(x_user, x_item, ei_rates, ei_rated_by, ei_similar,
           Wp_user, bp_user, Wp_item, bp_item,
           Wl_rates, bl_rates, Wr_rates,
           Wl_rb, bl_rb, Wr_rb,
           Wl_sim, bl_sim, Wr_sim,
           Wa1, ba1, Wa2):
    NREAL, D = x_user.shape
    H = Wp_user.shape[0]
    Hh = H // 2
    RB = 512
    nblk = -(-NREAL // RB)
    NA = nblk * RB
    NS = 16
    K = 120
    E = ei_rates.shape[1]
    CH = -(-E // (NS * K))
    Epad = CH * NS * K
    E_pw = Epad // NS
    SL = NA // NS

    f32 = _F32

    pe = Epad - E
    epad = jnp.concatenate(
        [jnp.zeros((1, pe), jnp.int32), jnp.full((1, pe), NREAL, jnp.int32)],
        axis=0)

    def edges(ei):
        return jnp.concatenate([ei.astype(jnp.int32), epad],
                               axis=1).reshape(2 * Epad)

    ei_r = edges(ei_rates)
    ei_b = edges(ei_rated_by)
    ei_s = edges(ei_similar)

    WpuT = Wp_user.T.astype(f32)
    WpiT = Wp_item.T.astype(f32)
    bpu = bp_user.reshape(1, H).astype(f32)
    bpi = bp_item.reshape(1, H).astype(f32)
    WrrT = Wr_rates.T.astype(f32)
    WrbT = Wr_rb.T.astype(f32)
    WrsT = Wr_sim.T.astype(f32)

    full = lambda shape: pl.BlockSpec(shape, lambda i: tuple(0 for _ in shape))
    rowblk = lambda w: pl.BlockSpec((RB, w), lambda i: (i, 0))

    xu_lo, xu_hi, xi_lo, xi_hi, rr, rb, rs = pl.pallas_call(
        _proj_body,
        grid=(nblk,),
        in_specs=[rowblk(D), rowblk(D),
                  full((D, H)), full((1, H)), full((D, H)), full((1, H)),
                  full((H, H)), full((H, H)), full((H, H))],
        out_specs=[rowblk(Hh), rowblk(Hh), rowblk(Hh), rowblk(Hh),
                   rowblk(H), rowblk(H), rowblk(H)],
        out_shape=[jax.ShapeDtypeStruct((NA, Hh), f32),
                   jax.ShapeDtypeStruct((NA, Hh), f32),
                   jax.ShapeDtypeStruct((NA, Hh), f32),
                   jax.ShapeDtypeStruct((NA, Hh), f32),
                   jax.ShapeDtypeStruct((NA, H), f32),
                   jax.ShapeDtypeStruct((NA, H), f32),
                   jax.ShapeDtypeStruct((NA, H), f32)],
    )(x_user.astype(f32), x_item.astype(f32),
      WpuT, bpu, WpiT, bpi, WrrT, WrbT, WrsT)

    z2 = jnp.zeros((NA, Hh), f32)
    z1 = jnp.zeros((NA,), f32)
    segsum = _make_segsum(NA, SL, E_pw, CH, K, Hh)

    slo_r, shi_r, cnt_r = segsum(xu_lo, xu_hi, ei_r, z2, z1)
    slo_b, shi_b, cnt_b = segsum(xi_lo, xi_hi, ei_b, z2, z1)
    slo_s, shi_s, cnt_s = segsum(xi_lo, xi_hi, ei_s, z2, z1)

    WlrT = Wl_rates.T.astype(f32)
    WlbT = Wl_rb.T.astype(f32)
    WlsT = Wl_sim.T.astype(f32)
    blr = bl_rates.reshape(1, H).astype(f32)
    blb = bl_rb.reshape(1, H).astype(f32)
    bls = bl_sim.reshape(1, H).astype(f32)
    Wa1T = Wa1.T.astype(f32)
    ba1r = ba1.reshape(1, H).astype(f32)
    Wa2T = Wa2.T.astype(f32)

    out_user, m_rates_f, m_sim_f, acc = pl.pallas_call(
        functools.partial(_post_body, NREAL),
        grid=(nblk,),
        in_specs=[rowblk(Hh), rowblk(Hh), rowblk(1),
                  rowblk(Hh), rowblk(Hh), rowblk(1),
                  rowblk(Hh), rowblk(Hh), rowblk(1),
                  rowblk(H), rowblk(H), rowblk(H),
                  full((Hh, H)), full((Hh, H)), full((1, H)),
                  full((Hh, H)), full((Hh, H)), full((1, H)),
                  full((Hh, H)), full((Hh, H)), full((1, H)),
                  full((H, H)), full((1, H)), full((H, 1))],
        out_specs=[rowblk(H), rowblk(H), rowblk(H),
                   pl.BlockSpec((8, H), lambda i: (0, 0))],
        out_shape=[jax.ShapeDtypeStruct((NREAL, H), f32),
                   jax.ShapeDtypeStruct((NA, H), f32),
                   jax.ShapeDtypeStruct((NA, H), f32),
                   jax.ShapeDtypeStruct((8, H), f32)],
    )(slo_r, shi_r, cnt_r.reshape(NA, 1),
      slo_b, shi_b, cnt_b.reshape(NA, 1),
      slo_s, shi_s, cnt_s.reshape(NA, 1),
      rr, rb, rs,
      WlrT[:Hh], WlrT[Hh:], blr, WlbT[:Hh], WlbT[Hh:], blb,
      WlsT[:Hh], WlsT[Hh:], bls, Wa1T, ba1r, Wa2T)

    out_item, pooled2 = pl.pallas_call(
        functools.partial(_combine_body, float(NREAL)),
        grid=(nblk,),
        in_specs=[rowblk(H), rowblk(H), pl.BlockSpec((8, H), lambda i: (0, 0))],
        out_specs=[rowblk(H), pl.BlockSpec((2, H), lambda i: (0, 0))],
        out_shape=[jax.ShapeDtypeStruct((NREAL, H), f32),
                   jax.ShapeDtypeStruct((2, H), f32)],
    )(m_rates_f, m_sim_f, acc)

    pooled = jnp.concatenate([pooled2[0], pooled2[1]], axis=0)
    return (pooled, out_user, out_item)


# speedup vs baseline: 1.1565x; 1.0920x over previous
import functools

import jax
import jax.numpy as jnp
from jax import lax
from jax.experimental import pallas as pl
from jax.experimental.pallas import tpu as pltpu
from jax.experimental.pallas import tpu_sc as plsc

_F32 = jnp.float32


def _dot(a, b):
    return jnp.dot(a, b, preferred_element_type=_F32)


def _proj_body(xu_ref, xi_ref, WpuT_ref, bpu_ref, WpiT_ref, bpi_ref,
               WrrT_ref, WrbT_ref, WrsT_ref,
               xu_lo_ref, xu_hi_ref, xi_lo_ref, xi_hi_ref,
               rr_ref, rb_ref, rs_ref):
    xu = jnp.maximum(_dot(xu_ref[...], WpuT_ref[...]) + bpu_ref[...], 0.0)
    xi = jnp.maximum(_dot(xi_ref[...], WpiT_ref[...]) + bpi_ref[...], 0.0)
    Hh = xu.shape[1] // 2
    xu_lo_ref[...] = xu[:, :Hh]
    xu_hi_ref[...] = xu[:, Hh:]
    xi_lo_ref[...] = xi[:, :Hh]
    xi_hi_ref[...] = xi[:, Hh:]
    rr_ref[...] = _dot(xi, WrrT_ref[...])
    rb_ref[...] = _dot(xu, WrbT_ref[...])
    rs_ref[...] = _dot(xi, WrsT_ref[...])


def _make_segsum(NA, SL, E_pw, CH, K, Hh):
    mesh = plsc.VectorSubcoreMesh(core_axis_name="c", subcore_axis_name="s")

    @functools.partial(
        pl.kernel,
        out_type=(
            jax.ShapeDtypeStruct((NA, Hh), _F32),
            jax.ShapeDtypeStruct((NA, Hh), _F32),
            jax.ShapeDtypeStruct((NA,), _F32),
        ),
        mesh=mesh,
        scratch_types=[
            pltpu.VMEM((K,), jnp.int32),
            pltpu.VMEM((K,), jnp.int32),
            pltpu.VMEM((K,), jnp.int32),
            pltpu.VMEM((K,), jnp.int32),
            pltpu.VMEM((K,), jnp.int32),
            pltpu.VMEM((K,), jnp.int32),
            pltpu.VMEM((K, Hh), _F32),
            pltpu.VMEM((K, Hh), _F32),
            pltpu.VMEM((K, Hh), _F32),
            pltpu.VMEM((128,), _F32),
            pltpu.VMEM_SHARED((NA, Hh), _F32),
            pltpu.VMEM_SHARED((NA,), _F32),
            pltpu.SemaphoreType.DMA,
            pltpu.SemaphoreType.DMA,
            pltpu.SemaphoreType.DMA,
            pltpu.SemaphoreType.DMA,
            pltpu.SemaphoreType.DMA,
            pltpu.SemaphoreType.DMA,
            pltpu.SemaphoreType.DMA,
            pltpu.SemaphoreType.DMA,
            pltpu.SemaphoreType.DMA,
        ],
    )
    def k(lo_hbm, hi_hbm, ei_hbm, z2_hbm, z1_hbm,
          sum_lo_hbm, sum_hi_hbm, cnt_hbm,
          sidx0, sidx1, sidx2, didx0, didx1, didx2,
          rows0, rows1, rows2, ones_pad,
          accum, cacc, g0, g1, g2, s0, s1, s2, d0, d1, d2):
        c = lax.axis_index("c")
        s = lax.axis_index("s")
        sidx = (sidx0, sidx1, sidx2)
        didx = (didx0, didx1, didx2)
        rows = (rows0, rows1, rows2)
        gsem = (g0, g1, g2)
        ssem = (s0, s1, s2)
        isem = (d0, d1, d2)
        ones = ones_pad.at[pl.ds(0, K)]
        rbase = s * SL
        pltpu.sync_copy(z2_hbm.at[pl.ds(rbase, SL)], accum.at[pl.ds(rbase, SL)])
        @pl.when(c == 0)
        def _():
            pltpu.sync_copy(z1_hbm.at[pl.ds(rbase, SL)],
                            cacc.at[pl.ds(rbase, SL)])
        for j in range(128 // 16):
            ones_pad[pl.ds(j * 16, 16)] = jnp.full((16,), 1.0, _F32)
        plsc.subcore_barrier()

        ebase = s * E_pw
        Epad = E_pw * 16

        def gstart(b):
            @pl.when(c == 0)
            def _():
                pltpu.async_copy(lo_hbm.at[sidx[b]], rows[b], gsem[b])

            @pl.when(c == 1)
            def _():
                pltpu.async_copy(hi_hbm.at[sidx[b]], rows[b], gsem[b])

        def gwait(b):
            @pl.when(c == 0)
            def _():
                pltpu.make_async_copy(lo_hbm.at[sidx[b]], rows[b],
                                      gsem[b]).wait()

            @pl.when(c == 1)
            def _():
                pltpu.make_async_copy(hi_hbm.at[sidx[b]], rows[b],
                                      gsem[b]).wait()

        def sstart(i, b):
            pltpu.async_copy(rows[b], accum.at[didx[b]], ssem[b], add=True)

            @pl.when(c == 0)
            def _():
                pltpu.async_copy(ones, cacc.at[didx[b]], ssem[b], add=True)

        def swait(i, b):
            pltpu.make_async_copy(rows[b], accum.at[didx[b]],
                                  ssem[b]).wait()

            @pl.when(c == 0)
            def _():
                pltpu.make_async_copy(ones, cacc.at[didx[b]],
                                      ssem[b]).wait()

        def dstart(i, b):
            off = ebase + i * K
            pltpu.async_copy(ei_hbm.at[pl.ds(Epad + off, K)], didx[b],
                             isem[b])

        def dwait(i, b):
            off = ebase + i * K
            pltpu.make_async_copy(ei_hbm.at[pl.ds(Epad + off, K)], didx[b],
                                  isem[b]).wait()

        for t in range(min(2, CH)):
            off = ebase + t * K
            pltpu.sync_copy(ei_hbm.at[pl.ds(off, K)], sidx[t])
            dstart(t, t)
            gstart(t)

        def consume(i, b):
            gwait(b)
            dwait(i, b)
            sstart(i, b)
            b2 = (b + 2) % 3

            @pl.when(i + 2 < CH)
            def _():
                off = ebase + (i + 2) * K
                pltpu.sync_copy(ei_hbm.at[pl.ds(off, K)], sidx[b2])

                @pl.when(i - 1 >= 0)
                def _():
                    swait(i - 1, b2)

                dstart(i + 2, b2)
                gstart(b2)

        def triple(j, carry):
            i0 = 3 * j
            for t in range(3):
                @pl.when(i0 + t < CH)
                def _(t=t):
                    consume(i0 + t, t)

            return carry

        lax.fori_loop(0, (CH + 2) // 3, triple, 0)

        for t in range(3):
            if CH > t:
                i_last = CH - 1 - ((CH - 1 - t) % 3)
                swait(i_last, t)

        plsc.subcore_barrier()

        @pl.when(c == 0)
        def _():
            pltpu.sync_copy(accum.at[pl.ds(rbase, SL)],
                            sum_lo_hbm.at[pl.ds(rbase, SL)])
            pltpu.sync_copy(cacc.at[pl.ds(rbase, SL)],
                            cnt_hbm.at[pl.ds(rbase, SL)])

        @pl.when(c == 1)
        def _():
            pltpu.sync_copy(accum.at[pl.ds(rbase, SL)],
                            sum_hi_hbm.at[pl.ds(rbase, SL)])

    return k


def _post_body(NREAL,
               slo_r_ref, shi_r_ref, cnt_r_ref,
               slo_b_ref, shi_b_ref, cnt_b_ref,
               slo_s_ref, shi_s_ref, cnt_s_ref,
               rr_ref, rb_ref, rs_ref,
               WlrT_lo_ref, WlrT_hi_ref, blr_ref,
               WlbT_lo_ref, WlbT_hi_ref, blb_ref,
               WlsT_lo_ref, WlsT_hi_ref, bls_ref,
               Wa1T_ref, ba1_ref, Wa2T_ref,
               out_user_ref, m_rates_ref, m_sim_ref, acc_ref):
    i = pl.program_id(0)
    RB = m_rates_ref.shape[0]

    def sage(slo_ref, shi_ref, cnt_ref, wt_lo_ref, wt_hi_ref, bl_ref, r_ref):
        t = _dot(slo_ref[...], wt_lo_ref[...]) + _dot(shi_ref[...], wt_hi_ref[...])
        inv = 1.0 / jnp.maximum(cnt_ref[...], 1.0)
        return jnp.maximum(t * inv + bl_ref[...] + r_ref[...], 0.0)

    m_rates = sage(slo_r_ref, shi_r_ref, cnt_r_ref, WlrT_lo_ref, WlrT_hi_ref,
                   blr_ref, rr_ref)
    m_rb = sage(slo_b_ref, shi_b_ref, cnt_b_ref, WlbT_lo_ref, WlbT_hi_ref,
                blb_ref, rb_ref)
    m_sim = sage(slo_s_ref, shi_s_ref, cnt_s_ref, WlsT_lo_ref, WlsT_hi_ref,
                 bls_ref, rs_ref)
    out_user_ref[...] = m_rb
    m_rates_ref[...] = m_rates
    m_sim_ref[...] = m_sim

    rows = lax.broadcasted_iota(jnp.int32, (RB, 1), 0) + i * RB
    valid = rows < NREAL

    a_rates = _dot(jnp.tanh(_dot(m_rates, Wa1T_ref[...]) + ba1_ref[...]),
                   Wa2T_ref[...])
    a_sim = _dot(jnp.tanh(_dot(m_sim, Wa1T_ref[...]) + ba1_ref[...]),
                 Wa2T_ref[...])

    cs_rb = jnp.sum(jnp.where(valid, m_rb, 0.0), axis=0, keepdims=True)
    cs_rates = jnp.sum(jnp.where(valid, m_rates, 0.0), axis=0, keepdims=True)
    cs_sim = jnp.sum(jnp.where(valid, m_sim, 0.0), axis=0, keepdims=True)
    wa = jnp.sum(jnp.where(valid, a_rates, 0.0))
    ws = jnp.sum(jnp.where(valid, a_sim, 0.0))
    col = lax.broadcasted_iota(jnp.int32, (1, cs_rb.shape[1]), 1)
    wrow = jnp.where(col == 0, wa, 0.0) + jnp.where(col == 1, ws, 0.0)
    upd = jnp.concatenate(
        [cs_rb, cs_rates, cs_sim, wrow,
         jnp.zeros((4, cs_rb.shape[1]), _F32)], axis=0)

    @pl.when(i == 0)
    def _():
        acc_ref[...] = jnp.zeros_like(acc_ref)

    acc_ref[...] += upd


def _combine_body(NREAL, m_rates_ref, m_sim_ref, acc_ref,
                  out_item_ref, pooled_ref):
    i = pl.program_id(0)
    acc = acc_ref[...]
    H = acc.shape[1]
    col = lax.broadcasted_iota(jnp.int32, (1, H), 1)
    wrow = acc[3:4, :]
    wa = jnp.sum(jnp.where(col == 0, wrow, 0.0)) / NREAL
    ws = jnp.sum(jnp.where(col == 1, wrow, 0.0)) / NREAL
    m = jnp.maximum(wa, ws)
    ea = jnp.exp(wa - m)
    es = jnp.exp(ws - m)
    tot = ea + es
    b0 = ea / tot
    b1 = es / tot
    out_item_ref[...] = b0 * m_rates_ref[...] + b1 * m_sim_ref[...]

    @pl.when(i == 0)
    def _():
        pooled_ref[...] = jnp.concatenate(
            [acc[0:1, :], b0 * acc[1:2, :] + b1 * acc[2:3, :]],
            axis=0) / NREAL


def kernel(x_user, x_item, ei_rates, ei_rated_by, ei_similar,
           Wp_user, bp_user, Wp_item, bp_item,
           Wl_rates, bl_rates, Wr_rates,
           Wl_rb, bl_rb, Wr_rb,
           Wl_sim, bl_sim, Wr_sim,
           Wa1, ba1, Wa2):
    NREAL, D = x_user.shape
    H = Wp_user.shape[0]
    Hh = H // 2
    RB = 512
    nblk = -(-NREAL // RB)
    NA = nblk * RB
    NS = 16
    K = 120
    E = ei_rates.shape[1]
    CH = -(-E // (NS * K))
    Epad = CH * NS * K
    E_pw = Epad // NS
    SL = NA // NS

    f32 = _F32

    pe = Epad - E
    epad = jnp.concatenate(
        [jnp.zeros((1, pe), jnp.int32), jnp.full((1, pe), NREAL, jnp.int32)],
        axis=0)

    def edges(ei):
        return jnp.concatenate([ei.astype(jnp.int32), epad],
                               axis=1).reshape(2 * Epad)

    ei_r = edges(ei_rates)
    ei_b = edges(ei_rated_by)
    ei_s = edges(ei_similar)

    WpuT = Wp_user.T.astype(f32)
    WpiT = Wp_item.T.astype(f32)
    bpu = bp_user.reshape(1, H).astype(f32)
    bpi = bp_item.reshape(1, H).astype(f32)
    WrrT = Wr_rates.T.astype(f32)
    WrbT = Wr_rb.T.astype(f32)
    WrsT = Wr_sim.T.astype(f32)

    full = lambda shape: pl.BlockSpec(shape, lambda i: tuple(0 for _ in shape))
    rowblk = lambda w: pl.BlockSpec((RB, w), lambda i: (i, 0))

    xu_lo, xu_hi, xi_lo, xi_hi, rr, rb, rs = pl.pallas_call(
        _proj_body,
        grid=(nblk,),
        in_specs=[rowblk(D), rowblk(D),
                  full((D, H)), full((1, H)), full((D, H)), full((1, H)),
                  full((H, H)), full((H, H)), full((H, H))],
        out_specs=[rowblk(Hh), rowblk(Hh), rowblk(Hh), rowblk(Hh),
                   rowblk(H), rowblk(H), rowblk(H)],
        out_shape=[jax.ShapeDtypeStruct((NA, Hh), f32),
                   jax.ShapeDtypeStruct((NA, Hh), f32),
                   jax.ShapeDtypeStruct((NA, Hh), f32),
                   jax.ShapeDtypeStruct((NA, Hh), f32),
                   jax.ShapeDtypeStruct((NA, H), f32),
                   jax.ShapeDtypeStruct((NA, H), f32),
                   jax.ShapeDtypeStruct((NA, H), f32)],
    )(x_user.astype(f32), x_item.astype(f32),
      WpuT, bpu, WpiT, bpi, WrrT, WrbT, WrsT)

    z2 = jnp.zeros((NA, Hh), f32)
    z1 = jnp.zeros((NA,), f32)
    segsum = _make_segsum(NA, SL, E_pw, CH, K, Hh)

    slo_r, shi_r, cnt_r = segsum(xu_lo, xu_hi, ei_r, z2, z1)
    slo_b, shi_b, cnt_b = segsum(xi_lo, xi_hi, ei_b, z2, z1)
    slo_s, shi_s, cnt_s = segsum(xi_lo, xi_hi, ei_s, z2, z1)

    WlrT = Wl_rates.T.astype(f32)
    WlbT = Wl_rb.T.astype(f32)
    WlsT = Wl_sim.T.astype(f32)
    blr = bl_rates.reshape(1, H).astype(f32)
    blb = bl_rb.reshape(1, H).astype(f32)
    bls = bl_sim.reshape(1, H).astype(f32)
    Wa1T = Wa1.T.astype(f32)
    ba1r = ba1.reshape(1, H).astype(f32)
    Wa2T = Wa2.T.astype(f32)

    out_user, m_rates_f, m_sim_f, acc = pl.pallas_call(
        functools.partial(_post_body, NREAL),
        grid=(nblk,),
        in_specs=[rowblk(Hh), rowblk(Hh), rowblk(1),
                  rowblk(Hh), rowblk(Hh), rowblk(1),
                  rowblk(Hh), rowblk(Hh), rowblk(1),
                  rowblk(H), rowblk(H), rowblk(H),
                  full((Hh, H)), full((Hh, H)), full((1, H)),
                  full((Hh, H)), full((Hh, H)), full((1, H)),
                  full((Hh, H)), full((Hh, H)), full((1, H)),
                  full((H, H)), full((1, H)), full((H, 1))],
        out_specs=[rowblk(H), rowblk(H), rowblk(H),
                   pl.BlockSpec((8, H), lambda i: (0, 0))],
        out_shape=[jax.ShapeDtypeStruct((NREAL, H), f32),
                   jax.ShapeDtypeStruct((NA, H), f32),
                   jax.ShapeDtypeStruct((NA, H), f32),
                   jax.ShapeDtypeStruct((8, H), f32)],
    )(slo_r, shi_r, cnt_r.reshape(NA, 1),
      slo_b, shi_b, cnt_b.reshape(NA, 1),
      slo_s, shi_s, cnt_s.reshape(NA, 1),
      rr, rb, rs,
      WlrT[:Hh], WlrT[Hh:], blr, WlbT[:Hh], WlbT[Hh:], blb,
      WlsT[:Hh], WlsT[Hh:], bls, Wa1T, ba1r, Wa2T)

    out_item, pooled2 = pl.pallas_call(
        functools.partial(_combine_body, float(NREAL)),
        grid=(nblk,),
        in_specs=[rowblk(H), rowblk(H), pl.BlockSpec((8, H), lambda i: (0, 0))],
        out_specs=[rowblk(H), pl.BlockSpec((2, H), lambda i: (0, 0))],
        out_shape=[jax.ShapeDtypeStruct((NREAL, H), f32),
                   jax.ShapeDtypeStruct((2, H), f32)],
    )(m_rates_f, m_sim_f, acc)

    pooled = jnp.concatenate([pooled2[0], pooled2[1]], axis=0)
    return (pooled, out_user, out_item)


# speedup vs baseline: 1.2132x; 1.0490x over previous
import functools

import jax
import jax.numpy as jnp
from jax import lax
from jax.experimental import pallas as pl
from jax.experimental.pallas import tpu as pltpu
from jax.experimental.pallas import tpu_sc as plsc

_F32 = jnp.float32


def _dot(a, b):
    return jnp.dot(a, b, preferred_element_type=_F32)


def _proj_body(xu_ref, xi_ref, WpuT_ref, bpu_ref, WpiT_ref, bpi_ref,
               WrrT_ref, WrbT_ref, WrsT_ref,
               xu_lo_ref, xu_hi_ref, xi_lo_ref, xi_hi_ref,
               rr_ref, rb_ref, rs_ref):
    xu = jnp.maximum(_dot(xu_ref[...], WpuT_ref[...]) + bpu_ref[...], 0.0)
    xi = jnp.maximum(_dot(xi_ref[...], WpiT_ref[...]) + bpi_ref[...], 0.0)
    Hh = xu.shape[1] // 2
    xu_lo_ref[...] = xu[:, :Hh]
    xu_hi_ref[...] = xu[:, Hh:]
    xi_lo_ref[...] = xi[:, :Hh]
    xi_hi_ref[...] = xi[:, Hh:]
    rr_ref[...] = _dot(xi, WrrT_ref[...])
    rb_ref[...] = _dot(xu, WrbT_ref[...])
    rs_ref[...] = _dot(xi, WrsT_ref[...])


def _make_segsum(NA, SL, E_pw, CH, K, Hh):
    mesh = plsc.VectorSubcoreMesh(core_axis_name="c", subcore_axis_name="s")

    @functools.partial(
        pl.kernel,
        out_type=(
            jax.ShapeDtypeStruct((NA, Hh), _F32),
            jax.ShapeDtypeStruct((NA, Hh), _F32),
            jax.ShapeDtypeStruct((NA,), _F32),
        ),
        mesh=mesh,
        scratch_types=[
            pltpu.VMEM((K,), jnp.int32),
            pltpu.VMEM((K,), jnp.int32),
            pltpu.VMEM((K,), jnp.int32),
            pltpu.VMEM((K,), jnp.int32),
            pltpu.VMEM((K,), jnp.int32),
            pltpu.VMEM((K,), jnp.int32),
            pltpu.VMEM((K, Hh), _F32),
            pltpu.VMEM((K, Hh), _F32),
            pltpu.VMEM((K, Hh), _F32),
            pltpu.VMEM((128,), _F32),
            pltpu.VMEM_SHARED((NA, Hh), _F32),
            pltpu.VMEM_SHARED((NA,), _F32),
            pltpu.SemaphoreType.DMA,
            pltpu.SemaphoreType.DMA,
            pltpu.SemaphoreType.DMA,
            pltpu.SemaphoreType.DMA,
            pltpu.SemaphoreType.DMA,
            pltpu.SemaphoreType.DMA,
            pltpu.SemaphoreType.DMA,
            pltpu.SemaphoreType.DMA,
            pltpu.SemaphoreType.DMA,
            pltpu.SemaphoreType.DMA,
            pltpu.SemaphoreType.DMA,
            pltpu.SemaphoreType.DMA,
        ],
    )
    def k(lo_hbm, hi_hbm, ei_hbm, z2_hbm, z1_hbm,
          sum_lo_hbm, sum_hi_hbm, cnt_hbm,
          sidx0, sidx1, sidx2, didx0, didx1, didx2,
          rows0, rows1, rows2, ones_pad,
          accum, cacc, g0, g1, g2, s0, s1, s2, d0, d1, d2, x0, x1, x2):
        c = lax.axis_index("c")
        s = lax.axis_index("s")
        sidx = (sidx0, sidx1, sidx2)
        didx = (didx0, didx1, didx2)
        rows = (rows0, rows1, rows2)
        gsem = (g0, g1, g2)
        ssem = (s0, s1, s2)
        isem = (d0, d1, d2)
        xsem = (x0, x1, x2)
        ones = ones_pad.at[pl.ds(0, K)]
        rbase = s * SL
        pltpu.sync_copy(z2_hbm.at[pl.ds(rbase, SL)], accum.at[pl.ds(rbase, SL)])
        @pl.when(c == 0)
        def _():
            pltpu.sync_copy(z1_hbm.at[pl.ds(rbase, SL)],
                            cacc.at[pl.ds(rbase, SL)])
        for j in range(128 // 16):
            ones_pad[pl.ds(j * 16, 16)] = jnp.full((16,), 1.0, _F32)
        plsc.subcore_barrier()

        ebase = s * E_pw
        Epad = E_pw * 16

        def gstart(b):
            @pl.when(c == 0)
            def _():
                pltpu.async_copy(lo_hbm.at[sidx[b]], rows[b], gsem[b])

            @pl.when(c == 1)
            def _():
                pltpu.async_copy(hi_hbm.at[sidx[b]], rows[b], gsem[b])

        def gwait(b):
            @pl.when(c == 0)
            def _():
                pltpu.make_async_copy(lo_hbm.at[sidx[b]], rows[b],
                                      gsem[b]).wait()

            @pl.when(c == 1)
            def _():
                pltpu.make_async_copy(hi_hbm.at[sidx[b]], rows[b],
                                      gsem[b]).wait()

        def sstart(i, b):
            pltpu.async_copy(rows[b], accum.at[didx[b]], ssem[b], add=True)

            @pl.when(c == 0)
            def _():
                pltpu.async_copy(ones, cacc.at[didx[b]], ssem[b], add=True)

        def swait(i, b):
            pltpu.make_async_copy(rows[b], accum.at[didx[b]],
                                  ssem[b]).wait()

            @pl.when(c == 0)
            def _():
                pltpu.make_async_copy(ones, cacc.at[didx[b]],
                                      ssem[b]).wait()

        def dstart(i, b):
            off = ebase + i * K
            pltpu.async_copy(ei_hbm.at[pl.ds(Epad + off, K)], didx[b],
                             isem[b])

        def dwait(i, b):
            off = ebase + i * K
            pltpu.make_async_copy(ei_hbm.at[pl.ds(Epad + off, K)], didx[b],
                                  isem[b]).wait()

        def xstart(i, b):
            off = ebase + i * K
            pltpu.async_copy(ei_hbm.at[pl.ds(off, K)], sidx[b], xsem[b])

        def xwait(i, b):
            off = ebase + i * K
            pltpu.make_async_copy(ei_hbm.at[pl.ds(off, K)], sidx[b],
                                  xsem[b]).wait()

        for t in range(min(2, CH)):
            off = ebase + t * K
            pltpu.sync_copy(ei_hbm.at[pl.ds(off, K)], sidx[t])
            dstart(t, t)
            gstart(t)
        if CH > 2:
            xstart(2, 2)

        def consume(i, b):
            gwait(b)

            @pl.when(i + 3 < CH)
            def _():
                xstart(i + 3, b)

            dwait(i, b)
            sstart(i, b)
            b2 = (b + 2) % 3

            @pl.when(i + 2 < CH)
            def _():
                xwait(i + 2, b2)

                @pl.when(i - 1 >= 0)
                def _():
                    swait(i - 1, b2)

                dstart(i + 2, b2)
                gstart(b2)

        def triple(j, carry):
            i0 = 3 * j
            for t in range(3):
                @pl.when(i0 + t < CH)
                def _(t=t):
                    consume(i0 + t, t)

            return carry

        lax.fori_loop(0, (CH + 2) // 3, triple, 0)

        for t in range(3):
            if CH > t:
                i_last = CH - 1 - ((CH - 1 - t) % 3)
                swait(i_last, t)

        plsc.subcore_barrier()

        @pl.when(c == 0)
        def _():
            pltpu.sync_copy(accum.at[pl.ds(rbase, SL)],
                            sum_lo_hbm.at[pl.ds(rbase, SL)])
            pltpu.sync_copy(cacc.at[pl.ds(rbase, SL)],
                            cnt_hbm.at[pl.ds(rbase, SL)])

        @pl.when(c == 1)
        def _():
            pltpu.sync_copy(accum.at[pl.ds(rbase, SL)],
                            sum_hi_hbm.at[pl.ds(rbase, SL)])

    return k


def _post_body(NREAL,
               slo_r_ref, shi_r_ref, cnt_r_ref,
               slo_b_ref, shi_b_ref, cnt_b_ref,
               slo_s_ref, shi_s_ref, cnt_s_ref,
               rr_ref, rb_ref, rs_ref,
               WlrT_lo_ref, WlrT_hi_ref, blr_ref,
               WlbT_lo_ref, WlbT_hi_ref, blb_ref,
               WlsT_lo_ref, WlsT_hi_ref, bls_ref,
               Wa1T_ref, ba1_ref, Wa2T_ref,
               out_user_ref, m_rates_ref, m_sim_ref, acc_ref):
    i = pl.program_id(0)
    RB = m_rates_ref.shape[0]

    def sage(slo_ref, shi_ref, cnt_ref, wt_lo_ref, wt_hi_ref, bl_ref, r_ref):
        t = _dot(slo_ref[...], wt_lo_ref[...]) + _dot(shi_ref[...], wt_hi_ref[...])
        inv = 1.0 / jnp.maximum(cnt_ref[...], 1.0)
        return jnp.maximum(t * inv + bl_ref[...] + r_ref[...], 0.0)

    m_rates = sage(slo_r_ref, shi_r_ref, cnt_r_ref, WlrT_lo_ref, WlrT_hi_ref,
                   blr_ref, rr_ref)
    m_rb = sage(slo_b_ref, shi_b_ref, cnt_b_ref, WlbT_lo_ref, WlbT_hi_ref,
                blb_ref, rb_ref)
    m_sim = sage(slo_s_ref, shi_s_ref, cnt_s_ref, WlsT_lo_ref, WlsT_hi_ref,
                 bls_ref, rs_ref)
    out_user_ref[...] = m_rb
    m_rates_ref[...] = m_rates
    m_sim_ref[...] = m_sim

    rows = lax.broadcasted_iota(jnp.int32, (RB, 1), 0) + i * RB
    valid = rows < NREAL

    a_rates = _dot(jnp.tanh(_dot(m_rates, Wa1T_ref[...]) + ba1_ref[...]),
                   Wa2T_ref[...])
    a_sim = _dot(jnp.tanh(_dot(m_sim, Wa1T_ref[...]) + ba1_ref[...]),
                 Wa2T_ref[...])

    cs_rb = jnp.sum(jnp.where(valid, m_rb, 0.0), axis=0, keepdims=True)
    cs_rates = jnp.sum(jnp.where(valid, m_rates, 0.0), axis=0, keepdims=True)
    cs_sim = jnp.sum(jnp.where(valid, m_sim, 0.0), axis=0, keepdims=True)
    wa = jnp.sum(jnp.where(valid, a_rates, 0.0))
    ws = jnp.sum(jnp.where(valid, a_sim, 0.0))
    col = lax.broadcasted_iota(jnp.int32, (1, cs_rb.shape[1]), 1)
    wrow = jnp.where(col == 0, wa, 0.0) + jnp.where(col == 1, ws, 0.0)
    upd = jnp.concatenate(
        [cs_rb, cs_rates, cs_sim, wrow,
         jnp.zeros((4, cs_rb.shape[1]), _F32)], axis=0)

    @pl.when(i == 0)
    def _():
        acc_ref[...] = jnp.zeros_like(acc_ref)

    acc_ref[...] += upd


def _combine_body(NREAL, m_rates_ref, m_sim_ref, acc_ref,
                  out_item_ref, pooled_ref):
    i = pl.program_id(0)
    acc = acc_ref[...]
    H = acc.shape[1]
    col = lax.broadcasted_iota(jnp.int32, (1, H), 1)
    wrow = acc[3:4, :]
    wa = jnp.sum(jnp.where(col == 0, wrow, 0.0)) / NREAL
    ws = jnp.sum(jnp.where(col == 1, wrow, 0.0)) / NREAL
    m = jnp.maximum(wa, ws)
    ea = jnp.exp(wa - m)
    es = jnp.exp(ws - m)
    tot = ea + es
    b0 = ea / tot
    b1 = es / tot
    out_item_ref[...] = b0 * m_rates_ref[...] + b1 * m_sim_ref[...]

    @pl.when(i == 0)
    def _():
        pooled_ref[...] = jnp.concatenate(
            [acc[0:1, :], b0 * acc[1:2, :] + b1 * acc[2:3, :]],
            axis=0) / NREAL


def kernel(x_user, x_item, ei_rates, ei_rated_by, ei_similar,
           Wp_user, bp_user, Wp_item, bp_item,
           Wl_rates, bl_rates, Wr_rates,
           Wl_rb, bl_rb, Wr_rb,
           Wl_sim, bl_sim, Wr_sim,
           Wa1, ba1, Wa2):
    NREAL, D = x_user.shape
    H = Wp_user.shape[0]
    Hh = H // 2
    RB = 512
    nblk = -(-NREAL // RB)
    NA = nblk * RB
    NS = 16
    K = 120
    E = ei_rates.shape[1]
    CH = -(-E // (NS * K))
    Epad = CH * NS * K
    E_pw = Epad // NS
    SL = NA // NS

    f32 = _F32

    pe = Epad - E
    epad = jnp.concatenate(
        [jnp.zeros((1, pe), jnp.int32), jnp.full((1, pe), NREAL, jnp.int32)],
        axis=0)

    def edges(ei):
        return jnp.concatenate([ei.astype(jnp.int32), epad],
                               axis=1).reshape(2 * Epad)

    ei_r = edges(ei_rates)
    ei_b = edges(ei_rated_by)
    ei_s = edges(ei_similar)

    WpuT = Wp_user.T.astype(f32)
    WpiT = Wp_item.T.astype(f32)
    bpu = bp_user.reshape(1, H).astype(f32)
    bpi = bp_item.reshape(1, H).astype(f32)
    WrrT = Wr_rates.T.astype(f32)
    WrbT = Wr_rb.T.astype(f32)
    WrsT = Wr_sim.T.astype(f32)

    full = lambda shape: pl.BlockSpec(shape, lambda i: tuple(0 for _ in shape))
    rowblk = lambda w: pl.BlockSpec((RB, w), lambda i: (i, 0))

    xu_lo, xu_hi, xi_lo, xi_hi, rr, rb, rs = pl.pallas_call(
        _proj_body,
        grid=(nblk,),
        in_specs=[rowblk(D), rowblk(D),
                  full((D, H)), full((1, H)), full((D, H)), full((1, H)),
                  full((H, H)), full((H, H)), full((H, H))],
        out_specs=[rowblk(Hh), rowblk(Hh), rowblk(Hh), rowblk(Hh),
                   rowblk(H), rowblk(H), rowblk(H)],
        out_shape=[jax.ShapeDtypeStruct((NA, Hh), f32),
                   jax.ShapeDtypeStruct((NA, Hh), f32),
                   jax.ShapeDtypeStruct((NA, Hh), f32),
                   jax.ShapeDtypeStruct((NA, Hh), f32),
                   jax.ShapeDtypeStruct((NA, H), f32),
                   jax.ShapeDtypeStruct((NA, H), f32),
                   jax.ShapeDtypeStruct((NA, H), f32)],
    )(x_user.astype(f32), x_item.astype(f32),
      WpuT, bpu, WpiT, bpi, WrrT, WrbT, WrsT)

    z2 = jnp.zeros((NA, Hh), f32)
    z1 = jnp.zeros((NA,), f32)
    segsum = _make_segsum(NA, SL, E_pw, CH, K, Hh)

    slo_r, shi_r, cnt_r = segsum(xu_lo, xu_hi, ei_r, z2, z1)
    slo_b, shi_b, cnt_b = segsum(xi_lo, xi_hi, ei_b, z2, z1)
    slo_s, shi_s, cnt_s = segsum(xi_lo, xi_hi, ei_s, z2, z1)

    WlrT = Wl_rates.T.astype(f32)
    WlbT = Wl_rb.T.astype(f32)
    WlsT = Wl_sim.T.astype(f32)
    blr = bl_rates.reshape(1, H).astype(f32)
    blb = bl_rb.reshape(1, H).astype(f32)
    bls = bl_sim.reshape(1, H).astype(f32)
    Wa1T = Wa1.T.astype(f32)
    ba1r = ba1.reshape(1, H).astype(f32)
    Wa2T = Wa2.T.astype(f32)

    out_user, m_rates_f, m_sim_f, acc = pl.pallas_call(
        functools.partial(_post_body, NREAL),
        grid=(nblk,),
        in_specs=[rowblk(Hh), rowblk(Hh), rowblk(1),
                  rowblk(Hh), rowblk(Hh), rowblk(1),
                  rowblk(Hh), rowblk(Hh), rowblk(1),
                  rowblk(H), rowblk(H), rowblk(H),
                  full((Hh, H)), full((Hh, H)), full((1, H)),
                  full((Hh, H)), full((Hh, H)), full((1, H)),
                  full((Hh, H)), full((Hh, H)), full((1, H)),
                  full((H, H)), full((1, H)), full((H, 1))],
        out_specs=[rowblk(H), rowblk(H), rowblk(H),
                   pl.BlockSpec((8, H), lambda i: (0, 0))],
        out_shape=[jax.ShapeDtypeStruct((NREAL, H), f32),
                   jax.ShapeDtypeStruct((NA, H), f32),
                   jax.ShapeDtypeStruct((NA, H), f32),
                   jax.ShapeDtypeStruct((8, H), f32)],
    )(slo_r, shi_r, cnt_r.reshape(NA, 1),
      slo_b, shi_b, cnt_b.reshape(NA, 1),
      slo_s, shi_s, cnt_s.reshape(NA, 1),
      rr, rb, rs,
      WlrT[:Hh], WlrT[Hh:], blr, WlbT[:Hh], WlbT[Hh:], blb,
      WlsT[:Hh], WlsT[Hh:], bls, Wa1T, ba1r, Wa2T)

    out_item, pooled2 = pl.pallas_call(
        functools.partial(_combine_body, float(NREAL)),
        grid=(nblk,),
        in_specs=[rowblk(H), rowblk(H), pl.BlockSpec((8, H), lambda i: (0, 0))],
        out_specs=[rowblk(H), pl.BlockSpec((2, H), lambda i: (0, 0))],
        out_shape=[jax.ShapeDtypeStruct((NREAL, H), f32),
                   jax.ShapeDtypeStruct((2, H), f32)],
    )(m_rates_f, m_sim_f, acc)

    pooled = jnp.concatenate([pooled2[0], pooled2[1]], axis=0)
    return (pooled, out_user, out_item)
